# Initial kernel scaffold; baseline (speedup 1.0000x reference)
import jax
import jax.numpy as jnp
from jax.experimental import pallas as pl


def kernel(x, edge_index, W1, b1, W2, b2):
    raise NotImplementedError("write your pallas kernel here")


# speedup vs baseline: 2.6995x; 2.6995x over previous
import functools

import jax
import jax.numpy as jnp
from jax import lax
from jax.experimental import pallas as pl
from jax.experimental.pallas import tpu as pltpu
from jax.experimental.pallas import tpu_sc as plsc

N_NODES = 10000
D = 256
QTR = 64
NQ = 4
NC = 2
NS = 16
CHUNK = 128
N_CHUNKS = 80
E_PER_SUB = CHUNK * N_CHUNKS
E_PAD = E_PER_SUB * NS
ACC_ROWS = 10240
ZROWS = ACC_ROWS // NS


def _fill_zero(buf, nrows):
    lanes = QTR // 16

    def body(i, carry):
        r = i // lanes
        k = i % lanes
        buf[r, pl.ds(k * 16, 16)] = jnp.zeros((16,), jnp.float32)
        return carry
    lax.fori_loop(0, nrows * lanes, body, 0)


def _agg_body(x4_hbm, src4_hbm, dst_hbm, out_hbm, sidx, didx, rows, zbuf, acc, sem):
    c = lax.axis_index("c")
    s = lax.axis_index("s")

    _fill_zero(zbuf, CHUNK)

    pltpu.sync_copy(dst_hbm.at[s], didx)

    for p in range(2):
        def zacc(j, carry):
            pltpu.sync_copy(zbuf, acc.at[pl.ds(s * ZROWS + j * CHUNK, CHUNK)])
            return carry
        lax.fori_loop(0, ZROWS // CHUNK, zacc, 0)

        pltpu.sync_copy(src4_hbm.at[2 * c + p, s], sidx)

        plsc.subcore_barrier()

        def chunk(j, carry):
            pltpu.async_copy(x4_hbm.at[sidx.at[j]], rows, sem).wait()
            pltpu.sync_copy(rows, acc.at[didx.at[j]], add=True)
            return carry
        lax.fori_loop(0, N_CHUNKS, chunk, 0)

        plsc.subcore_barrier()

        pltpu.sync_copy(acc.at[pl.ds(s * ZROWS, ZROWS)],
                        out_hbm.at[2 * c + p, pl.ds(s * ZROWS, ZROWS)])

        plsc.subcore_barrier()


def _sc_aggregate(x4, src4, dst_r):
    mesh = plsc.VectorSubcoreMesh(core_axis_name="c", subcore_axis_name="s",
                                  num_cores=NC, num_subcores=NS)
    k = functools.partial(
        pl.kernel,
        out_type=jax.ShapeDtypeStruct((NQ, ACC_ROWS, QTR), jnp.float32),
        mesh=mesh,
        scratch_types=[
            pltpu.VMEM((N_CHUNKS, CHUNK), jnp.int32),
            pltpu.VMEM((N_CHUNKS, CHUNK), jnp.int32),
            pltpu.VMEM((CHUNK, QTR), jnp.float32),
            pltpu.VMEM((CHUNK, QTR), jnp.float32),
            pltpu.VMEM_SHARED((ACC_ROWS, QTR), jnp.float32),
            pltpu.SemaphoreType.DMA,
        ],
        compiler_params=pltpu.CompilerParams(use_tc_tiling_on_sc=False),
    )(_agg_body)
    return k(x4, src4, dst_r)


def _mlp_body(x_ref, a0_ref, a1_ref, a2_ref, a3_ref,
              w1_ref, b1_ref, w2_ref, b2_ref, o_ref):
    agg = jnp.concatenate([a0_ref[0], a1_ref[0], a2_ref[0], a3_ref[0]], axis=-1)
    h = x_ref[...] + agg
    h1 = jnp.dot(h, w1_ref[...], preferred_element_type=jnp.float32) + b1_ref[...]
    h1 = jnp.maximum(h1, 0.0)
    o_ref[...] = jnp.dot(h1, w2_ref[...], preferred_element_type=jnp.float32) + b2_ref[...]


def _mlp(x, agg, W1, b1, W2, b2):
    R = 1000
    grid = (N_NODES // R,)
    return pl.pallas_call(
        _mlp_body,
        grid=grid,
        in_specs=[
            pl.BlockSpec((R, D), lambda i: (i, 0)),
            pl.BlockSpec((1, R, QTR), lambda i: (0, i, 0)),
            pl.BlockSpec((1, R, QTR), lambda i: (1, i, 0)),
            pl.BlockSpec((1, R, QTR), lambda i: (2, i, 0)),
            pl.BlockSpec((1, R, QTR), lambda i: (3, i, 0)),
            pl.BlockSpec((D, 512), lambda i: (0, 0)),
            pl.BlockSpec((1, 512), lambda i: (0, 0)),
            pl.BlockSpec((512, D), lambda i: (0, 0)),
            pl.BlockSpec((1, D), lambda i: (0, 0)),
        ],
        out_specs=pl.BlockSpec((R, D), lambda i: (i, 0)),
        out_shape=jax.ShapeDtypeStruct((N_NODES, D), jnp.float32),
    )(x, agg, agg, agg, agg, W1, b1.reshape(1, 512), W2, b2.reshape(1, D))


def kernel(x, edge_index, W1, b1, W2, b2):
    src = edge_index[0]
    dst = edge_index[1]
    pad = E_PAD - src.shape[0]
    src_p = jnp.concatenate([src, jnp.zeros((pad,), jnp.int32)])
    dst_p = jnp.concatenate([dst, jnp.full((pad,), N_NODES, jnp.int32)])
    x4 = x.reshape(NQ * N_NODES, QTR)
    src4 = jnp.stack([src_p * 4 + q for q in range(NQ)]).reshape(
        NQ, NS, N_CHUNKS, CHUNK)
    dst_r = dst_p.reshape(NS, N_CHUNKS, CHUNK)

    agg = _sc_aggregate(x4, src4, dst_r)
    return _mlp(x, agg, W1, b1, W2, b2)


# speedup vs baseline: 3.4005x; 1.2597x over previous
import functools

import jax
import jax.numpy as jnp
from jax import lax
from jax.experimental import pallas as pl
from jax.experimental.pallas import tpu as pltpu
from jax.experimental.pallas import tpu_sc as plsc

N_NODES = 10000
D = 256
QTR = 64
NQ = 4
NC = 2
NS = 16
CHUNK = 128
N_CHUNKS = 80
E_PER_SUB = CHUNK * N_CHUNKS
E_PAD = E_PER_SUB * NS
ACC_ROWS = 10240
ZROWS = ACC_ROWS // NS


def _fill_zero(buf, nrows):
    lanes = QTR // 16

    def body(i, carry):
        r = i // lanes
        k = i % lanes
        buf[r, pl.ds(k * 16, 16)] = jnp.zeros((16,), jnp.float32)
        return carry
    lax.fori_loop(0, nrows * lanes, body, 0)


NBUF = 2


def _agg_body(x4_hbm, src4_hbm, dst_hbm, out_hbm, sidx, didx,
              rows0, rows1, zbuf, acc, sem0, sem1):
    c = lax.axis_index("c")
    s = lax.axis_index("s")
    rowbufs = (rows0, rows1)
    sems = (sem0, sem1)

    _fill_zero(zbuf, CHUNK)

    pltpu.sync_copy(dst_hbm.at[s], didx)

    for p in range(2):
        def zacc(j, carry):
            pltpu.sync_copy(zbuf, acc.at[pl.ds(s * ZROWS + j * CHUNK, CHUNK)])
            return carry
        lax.fori_loop(0, ZROWS // CHUNK, zacc, 0)

        pltpu.sync_copy(src4_hbm.at[2 * c + p, s], sidx)

        plsc.subcore_barrier()

        for b in range(NBUF):
            pltpu.async_copy(x4_hbm.at[sidx.at[b]], rowbufs[b], sems[b])

        def chunk_group(i, carry):
            j0 = i * NBUF
            for b in range(NBUF):
                j = j0 + b
                pltpu.make_async_copy(
                    x4_hbm.at[sidx.at[j]], rowbufs[b], sems[b]).wait()
                pltpu.sync_copy(rowbufs[b], acc.at[didx.at[j]], add=True)

                @pl.when(j + NBUF < N_CHUNKS)
                def _():
                    pltpu.async_copy(
                        x4_hbm.at[sidx.at[j + NBUF]], rowbufs[b], sems[b])
            return carry
        lax.fori_loop(0, N_CHUNKS // NBUF, chunk_group, 0)

        plsc.subcore_barrier()

        pltpu.sync_copy(acc.at[pl.ds(s * ZROWS, ZROWS)],
                        out_hbm.at[2 * c + p, pl.ds(s * ZROWS, ZROWS)])

        plsc.subcore_barrier()


def _sc_aggregate(x4, src4, dst_r):
    mesh = plsc.VectorSubcoreMesh(core_axis_name="c", subcore_axis_name="s",
                                  num_cores=NC, num_subcores=NS)
    k = functools.partial(
        pl.kernel,
        out_type=jax.ShapeDtypeStruct((NQ, ACC_ROWS, QTR), jnp.float32),
        mesh=mesh,
        scratch_types=[
            pltpu.VMEM((N_CHUNKS, CHUNK), jnp.int32),
            pltpu.VMEM((N_CHUNKS, CHUNK), jnp.int32),
            pltpu.VMEM((CHUNK, QTR), jnp.float32),
            pltpu.VMEM((CHUNK, QTR), jnp.float32),
            pltpu.VMEM((CHUNK, QTR), jnp.float32),
            pltpu.VMEM_SHARED((ACC_ROWS, QTR), jnp.float32),
            pltpu.SemaphoreType.DMA,
            pltpu.SemaphoreType.DMA,
        ],
        compiler_params=pltpu.CompilerParams(use_tc_tiling_on_sc=False),
    )(_agg_body)
    return k(x4, src4, dst_r)


def _mlp_body(x_ref, a0_ref, a1_ref, a2_ref, a3_ref,
              w1_ref, b1_ref, w2_ref, b2_ref, o_ref):
    agg = jnp.concatenate([a0_ref[0], a1_ref[0], a2_ref[0], a3_ref[0]], axis=-1)
    h = x_ref[...] + agg
    h1 = jnp.dot(h, w1_ref[...], preferred_element_type=jnp.float32) + b1_ref[...]
    h1 = jnp.maximum(h1, 0.0)
    o_ref[...] = jnp.dot(h1, w2_ref[...], preferred_element_type=jnp.float32) + b2_ref[...]


def _mlp(x, agg, W1, b1, W2, b2):
    R = 1000
    grid = (N_NODES // R,)
    return pl.pallas_call(
        _mlp_body,
        grid=grid,
        in_specs=[
            pl.BlockSpec((R, D), lambda i: (i, 0)),
            pl.BlockSpec((1, R, QTR), lambda i: (0, i, 0)),
            pl.BlockSpec((1, R, QTR), lambda i: (1, i, 0)),
            pl.BlockSpec((1, R, QTR), lambda i: (2, i, 0)),
            pl.BlockSpec((1, R, QTR), lambda i: (3, i, 0)),
            pl.BlockSpec((D, 512), lambda i: (0, 0)),
            pl.BlockSpec((1, 512), lambda i: (0, 0)),
            pl.BlockSpec((512, D), lambda i: (0, 0)),
            pl.BlockSpec((1, D), lambda i: (0, 0)),
        ],
        out_specs=pl.BlockSpec((R, D), lambda i: (i, 0)),
        out_shape=jax.ShapeDtypeStruct((N_NODES, D), jnp.float32),
    )(x, agg, agg, agg, agg, W1, b1.reshape(1, 512), W2, b2.reshape(1, D))


def kernel(x, edge_index, W1, b1, W2, b2):
    src = edge_index[0]
    dst = edge_index[1]
    pad = E_PAD - src.shape[0]
    src_p = jnp.concatenate([src, jnp.zeros((pad,), jnp.int32)])
    dst_p = jnp.concatenate([dst, jnp.full((pad,), N_NODES, jnp.int32)])
    x4 = x.reshape(NQ * N_NODES, QTR)
    src4 = jnp.stack([src_p * 4 + q for q in range(NQ)]).reshape(
        NQ, NS, N_CHUNKS, CHUNK)
    dst_r = dst_p.reshape(NS, N_CHUNKS, CHUNK)

    agg = _sc_aggregate(x4, src4, dst_r)
    return _mlp(x, agg, W1, b1, W2, b2)
